# baseline (speedup 1.0000x reference)
import functools

import jax
import jax.numpy as jnp
from jax import lax
from jax.experimental import pallas as pl
from jax.experimental.pallas import tpu as pltpu
from jax.experimental.pallas import tpu_sc as plsc

N = 10000
E = 320000
D_IN = 128
H1 = 32
H2 = 16

NC = 2
NS = 16
NW = NC * NS

CHUNK = 512
NBUF = 2
CHUNKS = 20
GROUPS = CHUNKS // NBUF
E_PER_TILE = CHUNKS * CHUNK
E_PAD = NW * E_PER_TILE
N_PAD = 10112
ROWS_PER_TILE = N_PAD // NS


def _make_segsum(D: int):
  mesh = plsc.VectorSubcoreMesh(
      core_axis_name="c", subcore_axis_name="s", num_cores=NC, num_subcores=NS
  )

  @functools.partial(
      pl.kernel,
      out_type=jax.ShapeDtypeStruct((NC * N_PAD, D), jnp.float32),
      mesh=mesh,
      compiler_params=pltpu.CompilerParams(use_tc_tiling_on_sc=False),
      scratch_types=[
          pltpu.VMEM((CHUNKS, CHUNK), jnp.int32),
          pltpu.VMEM((CHUNKS, CHUNK), jnp.int32),
          pltpu.VMEM_SHARED((N_PAD, D), jnp.float32),
      ] + [pltpu.VMEM((CHUNK, D), jnp.float32) for _ in range(NBUF)]
        + [pltpu.SemaphoreType.DMA for _ in range(NBUF)],
  )
  def seg(h_hbm, src_hbm, dst_hbm, zeros_hbm, out_hbm,
          src_v, dst_v, acc_sh, *rows_and_sems):
    rows = rows_and_sems[:NBUF]
    sems = rows_and_sems[NBUF:]
    cid = lax.axis_index("c")
    sid = lax.axis_index("s")
    wid = sid * NC + cid
    r0 = sid * ROWS_PER_TILE
    pltpu.sync_copy(zeros_hbm.at[pl.ds(r0, ROWS_PER_TILE)],
                    acc_sh.at[pl.ds(r0, ROWS_PER_TILE)])
    pltpu.sync_copy(src_hbm.at[wid], src_v)
    pltpu.sync_copy(dst_hbm.at[wid], dst_v)
    plsc.subcore_barrier()

    for b in range(NBUF):
      pltpu.async_copy(h_hbm.at[src_v.at[b]], rows[b], sems[b])

    def group(g, carry):
      for b in range(NBUF):
        j = g * NBUF + b
        pltpu.make_async_copy(h_hbm.at[src_v.at[j]], rows[b], sems[b]).wait()
        pltpu.sync_copy(rows[b], acc_sh.at[dst_v.at[j]], add=True)

        @pl.when(g < GROUPS - 1)
        def _():
          pltpu.async_copy(h_hbm.at[src_v.at[j + NBUF]], rows[b], sems[b])

      return carry

    lax.fori_loop(0, GROUPS, group, 0)
    plsc.subcore_barrier()
    pltpu.sync_copy(acc_sh.at[pl.ds(r0, ROWS_PER_TILE)],
                    out_hbm.at[pl.ds(cid * N_PAD + r0, ROWS_PER_TILE)])

  return seg


_segsum32 = _make_segsum(H1)
_segsum16 = _make_segsum(H2)


def _mm1_body(x_ref, w_ref, o_ref):
  o_ref[...] = jnp.dot(x_ref[...], w_ref[...],
                       preferred_element_type=jnp.float32)


def _mm2_body(p0_ref, p1_ref, w_ref, o_ref):
  h = jnp.maximum(p0_ref[...] + p1_ref[...], 0.0)
  o_ref[...] = jnp.dot(h, w_ref[...], preferred_element_type=jnp.float32)


BM = 400


def _dec_body(q0b_ref, q1b_ref, q0f_ref, q1f_ref, o_ref):
  zi = q0b_ref[...] + q1b_ref[...]
  zf = q0f_ref[...] + q1f_ref[...]
  o_ref[...] = lax.dot_general(zi, zf, (((1,), (1,)), ((), ())),
                               preferred_element_type=jnp.float32)


def kernel(x, edge_index, W1, W2):
  ei = edge_index.astype(jnp.int32)
  pad = E_PAD - E
  src = jnp.concatenate([ei[0], jnp.zeros((pad,), jnp.int32)])
  dst = jnp.concatenate([ei[1], jnp.full((pad,), N, jnp.int32)])
  src3 = src.reshape(NW, CHUNKS, CHUNK)
  dst3 = dst.reshape(NW, CHUNKS, CHUNK)
  zeros32 = jnp.zeros((N_PAD, H1), jnp.float32)
  zeros16 = jnp.zeros((N_PAD, H2), jnp.float32)

  h = pl.pallas_call(
      _mm1_body,
      out_shape=jax.ShapeDtypeStruct((N, H1), jnp.float32),
  )(x, W1)

  p = _segsum32(h, src3, dst3, zeros32)
  p0 = p[:N]
  p1 = p[N_PAD:N_PAD + N]

  h2 = pl.pallas_call(
      _mm2_body,
      out_shape=jax.ShapeDtypeStruct((N, H2), jnp.float32),
  )(p0, p1, W2)

  q = _segsum16(h2, src3, dst3, zeros16)
  q0 = q[:N]
  q1 = q[N_PAD:N_PAD + N]

  recon = pl.pallas_call(
      _dec_body,
      grid=(N // BM,),
      in_specs=[
          pl.BlockSpec((BM, H2), lambda i: (i, 0)),
          pl.BlockSpec((BM, H2), lambda i: (i, 0)),
          pl.BlockSpec((N, H2), lambda i: (0, 0)),
          pl.BlockSpec((N, H2), lambda i: (0, 0)),
      ],
      out_specs=pl.BlockSpec((BM, N), lambda i: (i, 0)),
      out_shape=jax.ShapeDtypeStruct((N, N), jnp.float32),
      compiler_params=pltpu.CompilerParams(
          dimension_semantics=("parallel",)),
  )(q0, q1, q0, q1)

  return recon.reshape(-1)


# speedup vs baseline: 1.0054x; 1.0054x over previous
import functools

import jax
import jax.numpy as jnp
from jax import lax
from jax.experimental import pallas as pl
from jax.experimental.pallas import tpu as pltpu
from jax.experimental.pallas import tpu_sc as plsc

N = 10000
E = 320000
D_IN = 128
H1 = 32
H2 = 16

NC = 2
NS = 16
NW = NC * NS

CHUNK = 128
NBUF = 8
CHUNKS = 80
GROUPS = CHUNKS // NBUF
E_PER_TILE = CHUNKS * CHUNK
E_PAD = NW * E_PER_TILE
N_PAD = 10112
ROWS_PER_TILE = N_PAD // NS


def _make_segsum(D: int):
  mesh = plsc.VectorSubcoreMesh(
      core_axis_name="c", subcore_axis_name="s", num_cores=NC, num_subcores=NS
  )

  @functools.partial(
      pl.kernel,
      out_type=jax.ShapeDtypeStruct((NC * N_PAD, D), jnp.float32),
      mesh=mesh,
      compiler_params=pltpu.CompilerParams(use_tc_tiling_on_sc=False),
      scratch_types=[
          pltpu.VMEM((CHUNKS, CHUNK), jnp.int32),
          pltpu.VMEM((CHUNKS, CHUNK), jnp.int32),
          pltpu.VMEM_SHARED((N_PAD, D), jnp.float32),
      ] + [pltpu.VMEM((CHUNK, D), jnp.float32) for _ in range(NBUF)]
        + [pltpu.SemaphoreType.DMA for _ in range(NBUF)],
  )
  def seg(h_hbm, src_hbm, dst_hbm, zeros_hbm, out_hbm,
          src_v, dst_v, acc_sh, *rows_and_sems):
    rows = rows_and_sems[:NBUF]
    sems = rows_and_sems[NBUF:]
    cid = lax.axis_index("c")
    sid = lax.axis_index("s")
    wid = sid * NC + cid
    r0 = sid * ROWS_PER_TILE
    pltpu.sync_copy(zeros_hbm.at[pl.ds(r0, ROWS_PER_TILE)],
                    acc_sh.at[pl.ds(r0, ROWS_PER_TILE)])
    pltpu.sync_copy(src_hbm.at[wid], src_v)
    pltpu.sync_copy(dst_hbm.at[wid], dst_v)
    plsc.subcore_barrier()

    for b in range(NBUF):
      pltpu.async_copy(h_hbm.at[src_v.at[b]], rows[b], sems[b])

    def group(g, carry):
      for b in range(NBUF):
        j = g * NBUF + b
        pltpu.make_async_copy(h_hbm.at[src_v.at[j]], rows[b], sems[b]).wait()
        pltpu.sync_copy(rows[b], acc_sh.at[dst_v.at[j]], add=True)

        @pl.when(g < GROUPS - 1)
        def _():
          pltpu.async_copy(h_hbm.at[src_v.at[j + NBUF]], rows[b], sems[b])

      return carry

    lax.fori_loop(0, GROUPS, group, 0)
    plsc.subcore_barrier()
    pltpu.sync_copy(acc_sh.at[pl.ds(r0, ROWS_PER_TILE)],
                    out_hbm.at[pl.ds(cid * N_PAD + r0, ROWS_PER_TILE)])

  return seg


_segsum32 = _make_segsum(H1)
_segsum16 = _make_segsum(H2)


def _mm1_body(x_ref, w_ref, o_ref):
  o_ref[...] = jnp.dot(x_ref[...], w_ref[...],
                       preferred_element_type=jnp.float32)


def _mm2_body(p0_ref, p1_ref, w_ref, o_ref):
  h = jnp.maximum(p0_ref[...] + p1_ref[...], 0.0)
  o_ref[...] = jnp.dot(h, w_ref[...], preferred_element_type=jnp.float32)


BM = 400


def _dec_body(q0b_ref, q1b_ref, q0f_ref, q1f_ref, o_ref):
  zi = q0b_ref[...] + q1b_ref[...]
  zf = q0f_ref[...] + q1f_ref[...]
  o_ref[...] = lax.dot_general(zi, zf, (((1,), (1,)), ((), ())),
                               preferred_element_type=jnp.float32)


def kernel(x, edge_index, W1, W2):
  ei = edge_index.astype(jnp.int32)
  pad = E_PAD - E
  src = jnp.concatenate([ei[0], jnp.zeros((pad,), jnp.int32)])
  dst = jnp.concatenate([ei[1], jnp.full((pad,), N, jnp.int32)])
  src3 = src.reshape(NW, CHUNKS, CHUNK)
  dst3 = dst.reshape(NW, CHUNKS, CHUNK)
  zeros32 = jnp.zeros((N_PAD, H1), jnp.float32)
  zeros16 = jnp.zeros((N_PAD, H2), jnp.float32)

  h = pl.pallas_call(
      _mm1_body,
      out_shape=jax.ShapeDtypeStruct((N, H1), jnp.float32),
  )(x, W1)

  p = _segsum32(h, src3, dst3, zeros32)
  p0 = p[:N]
  p1 = p[N_PAD:N_PAD + N]

  h2 = pl.pallas_call(
      _mm2_body,
      out_shape=jax.ShapeDtypeStruct((N, H2), jnp.float32),
  )(p0, p1, W2)

  q = _segsum16(h2, src3, dst3, zeros16)
  q0 = q[:N]
  q1 = q[N_PAD:N_PAD + N]

  recon = pl.pallas_call(
      _dec_body,
      grid=(N // BM,),
      in_specs=[
          pl.BlockSpec((BM, H2), lambda i: (i, 0)),
          pl.BlockSpec((BM, H2), lambda i: (i, 0)),
          pl.BlockSpec((N, H2), lambda i: (0, 0)),
          pl.BlockSpec((N, H2), lambda i: (0, 0)),
      ],
      out_specs=pl.BlockSpec((BM, N), lambda i: (i, 0)),
      out_shape=jax.ShapeDtypeStruct((N, N), jnp.float32),
      compiler_params=pltpu.CompilerParams(
          dimension_semantics=("parallel",)),
  )(q0, q1, q0, q1)

  return recon.reshape(-1)
